# Initial kernel scaffold; baseline (speedup 1.0000x reference)
import jax
import jax.numpy as jnp
from jax.experimental import pallas as pl


def kernel(x, edge_index, W1, b1, W2, b2):
    raise NotImplementedError("write your pallas kernel here")


# speedup vs baseline: 12.5659x; 12.5659x over previous
import functools

import jax
import jax.numpy as jnp
from jax import lax
from jax.experimental import pallas as pl
from jax.experimental.pallas import tpu as pltpu
from jax.experimental.pallas import tpu_sc as plsc

N = 10000
E = 320000
NC = 2
NS = 16
NW = NC * NS
EPW = E // NW
K = 80
NCHUNK = EPW // K
RB = 640
RB_LAST = N - 15 * RB


def _make_sc_accumulate(D):
  mesh = plsc.VectorSubcoreMesh(core_axis_name="c", subcore_axis_name="s")

  @functools.partial(
      pl.kernel,
      out_type=jax.ShapeDtypeStruct((NC * N, D), jnp.float32),
      mesh=mesh,
      compiler_params=pltpu.CompilerParams(use_tc_tiling_on_sc=False),
      scratch_types=[
          pltpu.VMEM((K,), jnp.int32),
          pltpu.VMEM((K,), jnp.int32),
          pltpu.VMEM((K, D), jnp.float32),
          pltpu.VMEM_SHARED((N, D), jnp.float32),
          pltpu.SemaphoreType.DMA,
      ],
  )
  def body(y_hbm, src_hbm, dst_hbm, zeros_hbm, out_hbm, src_v, dst_v,
           rows_v, acc_sh, sem):
    c = lax.axis_index("c")
    s = lax.axis_index("s")
    wid = c * NS + s
    r0 = s * RB

    @pl.when(s < NS - 1)
    def _():
      pltpu.sync_copy(zeros_hbm.at[pl.ds(r0, RB)], acc_sh.at[pl.ds(r0, RB)])

    @pl.when(s == NS - 1)
    def _():
      pltpu.sync_copy(zeros_hbm.at[pl.ds(r0, RB_LAST)],
                      acc_sh.at[pl.ds(r0, RB_LAST)])

    plsc.subcore_barrier()

    base0 = wid * EPW

    def step(i, carry):
      base = base0 + i * K
      pltpu.sync_copy(src_hbm.at[pl.ds(base, K)], src_v)
      pltpu.sync_copy(dst_hbm.at[pl.ds(base, K)], dst_v)
      pltpu.async_copy(y_hbm.at[src_v], rows_v, sem).wait()
      pltpu.sync_copy(rows_v, acc_sh.at[dst_v], add=True)
      return carry

    lax.fori_loop(0, NCHUNK, step, 0)
    plsc.subcore_barrier()

    @pl.when(s < NS - 1)
    def _():
      pltpu.sync_copy(acc_sh.at[pl.ds(r0, RB)],
                      out_hbm.at[pl.ds(c * N + r0, RB)])

    @pl.when(s == NS - 1)
    def _():
      pltpu.sync_copy(acc_sh.at[pl.ds(r0, RB_LAST)],
                      out_hbm.at[pl.ds(c * N + r0, RB_LAST)])

  return body


def _sc_accumulate(y, src, dst, D):
  zeros = jnp.zeros((N, D), jnp.float32)
  parts = _make_sc_accumulate(D)(y, src, dst, zeros)
  return parts[:N], parts[N:]


_GRID = 10
_BN = N // _GRID


def _mm_body(x_ref, w_ref, o_ref):
  o_ref[...] = jnp.dot(x_ref[...], w_ref[...],
                       preferred_element_type=jnp.float32)


def _tc_matmul(x, w):
  m, kdim = x.shape
  n = w.shape[1]
  return pl.pallas_call(
      _mm_body,
      grid=(_GRID,),
      in_specs=[
          pl.BlockSpec((_BN, kdim), lambda i: (i, 0)),
          pl.BlockSpec((kdim, n), lambda i: (0, 0)),
      ],
      out_specs=pl.BlockSpec((_BN, n), lambda i: (i, 0)),
      out_shape=jax.ShapeDtypeStruct((m, n), jnp.float32),
  )(x, w)


def _scale_body(p0_ref, p1_ref, xw_ref, dinv_ref, y_ref):
  deg = 1.0 + p0_ref[...] + p1_ref[...]
  dinv = lax.rsqrt(deg)
  dinv_ref[...] = dinv
  y_ref[...] = dinv * xw_ref[...]


def _tc_scale(p0, p1, xw):
  d = xw.shape[1]
  return pl.pallas_call(
      _scale_body,
      grid=(_GRID,),
      in_specs=[
          pl.BlockSpec((_BN, 1), lambda i: (i, 0)),
          pl.BlockSpec((_BN, 1), lambda i: (i, 0)),
          pl.BlockSpec((_BN, d), lambda i: (i, 0)),
      ],
      out_specs=[
          pl.BlockSpec((_BN, 1), lambda i: (i, 0)),
          pl.BlockSpec((_BN, d), lambda i: (i, 0)),
      ],
      out_shape=[
          jax.ShapeDtypeStruct((N, 1), jnp.float32),
          jax.ShapeDtypeStruct((N, d), jnp.float32),
      ],
  )(p0, p1, xw)


def _layer_body(a0_ref, a1_ref, y_ref, dinv_ref, b_ref, w_ref, o_ref):
  h = dinv_ref[...] * (a0_ref[...] + a1_ref[...] + y_ref[...]) + b_ref[...]
  h = jnp.maximum(h, 0.0)
  o_ref[...] = dinv_ref[...] * jnp.dot(h, w_ref[...],
                                       preferred_element_type=jnp.float32)


def _tc_layer(a0, a1, y, dinv, b, w):
  d = y.shape[1]
  n = w.shape[1]
  return pl.pallas_call(
      _layer_body,
      grid=(_GRID,),
      in_specs=[
          pl.BlockSpec((_BN, d), lambda i: (i, 0)),
          pl.BlockSpec((_BN, d), lambda i: (i, 0)),
          pl.BlockSpec((_BN, d), lambda i: (i, 0)),
          pl.BlockSpec((_BN, 1), lambda i: (i, 0)),
          pl.BlockSpec((1, d), lambda i: (0, 0)),
          pl.BlockSpec((d, n), lambda i: (0, 0)),
      ],
      out_specs=pl.BlockSpec((_BN, n), lambda i: (i, 0)),
      out_shape=jax.ShapeDtypeStruct((N, n), jnp.float32),
  )(a0, a1, y, dinv, b, w)


def _combine_body(a0_ref, a1_ref, y_ref, dinv_ref, b_ref, o_ref):
  o_ref[...] = (dinv_ref[...] * (a0_ref[...] + a1_ref[...] + y_ref[...])
                + b_ref[...])


def _tc_combine(a0, a1, y, dinv, b):
  d = y.shape[1]
  return pl.pallas_call(
      _combine_body,
      grid=(_GRID,),
      in_specs=[
          pl.BlockSpec((_BN, d), lambda i: (i, 0)),
          pl.BlockSpec((_BN, d), lambda i: (i, 0)),
          pl.BlockSpec((_BN, d), lambda i: (i, 0)),
          pl.BlockSpec((_BN, 1), lambda i: (i, 0)),
          pl.BlockSpec((1, d), lambda i: (0, 0)),
      ],
      out_specs=pl.BlockSpec((_BN, d), lambda i: (i, 0)),
      out_shape=jax.ShapeDtypeStruct((N, d), jnp.float32),
  )(a0, a1, y, dinv, b)


def kernel(x, edge_index, W1, b1, W2, b2):
  src = edge_index[0]
  dst = edge_index[1]

  ones = jnp.ones((N, 8), jnp.float32)
  d0, d1 = _sc_accumulate(ones, src, dst, 8)
  p0 = d0[:, 0:1]
  p1 = d1[:, 0:1]

  xw1 = _tc_matmul(x, W1)
  dinv, y1 = _tc_scale(p0, p1, xw1)

  a0, a1 = _sc_accumulate(y1, src, dst, 64)
  y2 = _tc_layer(a0, a1, y1, dinv, b1.reshape(1, -1), W2)

  c0, c1 = _sc_accumulate(y2, src, dst, 32)
  return _tc_combine(c0, c1, y2, dinv, b2.reshape(1, -1))


# speedup vs baseline: 30.0819x; 2.3939x over previous
import functools

import jax
import jax.numpy as jnp
from jax import lax
from jax.experimental import pallas as pl
from jax.experimental.pallas import tpu as pltpu
from jax.experimental.pallas import tpu_sc as plsc

N = 10000
E = 320000
NC = 2
NS = 16
NW = NC * NS
EPW = E // NW
K = 80
NCHUNK = EPW // K
RB = 640
RB_LAST = N - 15 * RB


def _make_sc_accumulate(D):
  mesh = plsc.VectorSubcoreMesh(core_axis_name="c", subcore_axis_name="s")

  @functools.partial(
      pl.kernel,
      out_type=jax.ShapeDtypeStruct((NC * N, D), jnp.float32),
      mesh=mesh,
      compiler_params=pltpu.CompilerParams(use_tc_tiling_on_sc=False),
      scratch_types=[
          pltpu.VMEM((NCHUNK, K), jnp.int32),
          pltpu.VMEM((NCHUNK, K), jnp.int32),
          pltpu.VMEM((K, D), jnp.float32),
          pltpu.VMEM((K, D), jnp.float32),
          pltpu.VMEM_SHARED((N, D), jnp.float32),
          pltpu.SemaphoreType.DMA,
          pltpu.SemaphoreType.DMA,
      ],
  )
  def body(y_hbm, src_hbm, dst_hbm, zeros_hbm, out_hbm, src_all, dst_all,
           rows0, rows1, acc_sh, sem0, sem1):
    c = lax.axis_index("c")
    s = lax.axis_index("s")
    wid = c * NS + s
    r0 = s * RB

    @pl.when(s < NS - 1)
    def _():
      pltpu.sync_copy(zeros_hbm.at[pl.ds(r0, RB)], acc_sh.at[pl.ds(r0, RB)])

    @pl.when(s == NS - 1)
    def _():
      pltpu.sync_copy(zeros_hbm.at[pl.ds(r0, RB_LAST)],
                      acc_sh.at[pl.ds(r0, RB_LAST)])

    pltpu.sync_copy(src_hbm.at[wid], src_all)
    pltpu.sync_copy(dst_hbm.at[wid], dst_all)
    plsc.subcore_barrier()

    pltpu.async_copy(y_hbm.at[src_all.at[0]], rows0, sem0)

    def pair(jj, carry):
      j = jj * 2
      pltpu.async_copy(y_hbm.at[src_all.at[j + 1]], rows1, sem1)
      pltpu.make_async_copy(y_hbm.at[src_all.at[j]], rows0, sem0).wait()
      pltpu.sync_copy(rows0, acc_sh.at[dst_all.at[j]], add=True)
      pltpu.async_copy(y_hbm.at[src_all.at[j + 2]], rows0, sem0)
      pltpu.make_async_copy(y_hbm.at[src_all.at[j + 1]], rows1, sem1).wait()
      pltpu.sync_copy(rows1, acc_sh.at[dst_all.at[j + 1]], add=True)
      return carry

    lax.fori_loop(0, NCHUNK // 2, pair, 0)
    pltpu.make_async_copy(y_hbm.at[src_all.at[NCHUNK - 1]], rows0,
                          sem0).wait()
    pltpu.sync_copy(rows0, acc_sh.at[dst_all.at[NCHUNK - 1]], add=True)
    plsc.subcore_barrier()

    @pl.when(s < NS - 1)
    def _():
      pltpu.sync_copy(acc_sh.at[pl.ds(r0, RB)],
                      out_hbm.at[pl.ds(c * N + r0, RB)])

    @pl.when(s == NS - 1)
    def _():
      pltpu.sync_copy(acc_sh.at[pl.ds(r0, RB_LAST)],
                      out_hbm.at[pl.ds(c * N + r0, RB_LAST)])

  return body


def _sc_accumulate(y, src, dst, D):
  zeros = jnp.zeros((N, D), jnp.float32)
  srcw = src.reshape(NW, NCHUNK, K)
  dstw = dst.reshape(NW, NCHUNK, K)
  parts = _make_sc_accumulate(D)(y, srcw, dstw, zeros)
  return parts[:N], parts[N:]


_GRID = 10
_BN = N // _GRID


def _mm_body(x_ref, w_ref, o_ref):
  o_ref[...] = jnp.dot(x_ref[...], w_ref[...],
                       preferred_element_type=jnp.float32)


def _tc_matmul(x, w):
  m, kdim = x.shape
  n = w.shape[1]
  return pl.pallas_call(
      _mm_body,
      grid=(_GRID,),
      in_specs=[
          pl.BlockSpec((_BN, kdim), lambda i: (i, 0)),
          pl.BlockSpec((kdim, n), lambda i: (0, 0)),
      ],
      out_specs=pl.BlockSpec((_BN, n), lambda i: (i, 0)),
      out_shape=jax.ShapeDtypeStruct((m, n), jnp.float32),
  )(x, w)


def _scale_body(p0_ref, p1_ref, xw_ref, dinv_ref, y_ref):
  deg = 1.0 + p0_ref[...] + p1_ref[...]
  dinv = lax.rsqrt(deg)
  dinv_ref[...] = dinv
  y_ref[...] = dinv * xw_ref[...]


def _tc_scale(p0, p1, xw):
  d = xw.shape[1]
  return pl.pallas_call(
      _scale_body,
      grid=(_GRID,),
      in_specs=[
          pl.BlockSpec((_BN, 1), lambda i: (i, 0)),
          pl.BlockSpec((_BN, 1), lambda i: (i, 0)),
          pl.BlockSpec((_BN, d), lambda i: (i, 0)),
      ],
      out_specs=[
          pl.BlockSpec((_BN, 1), lambda i: (i, 0)),
          pl.BlockSpec((_BN, d), lambda i: (i, 0)),
      ],
      out_shape=[
          jax.ShapeDtypeStruct((N, 1), jnp.float32),
          jax.ShapeDtypeStruct((N, d), jnp.float32),
      ],
  )(p0, p1, xw)


def _layer_body(a0_ref, a1_ref, y_ref, dinv_ref, b_ref, w_ref, o_ref):
  h = dinv_ref[...] * (a0_ref[...] + a1_ref[...] + y_ref[...]) + b_ref[...]
  h = jnp.maximum(h, 0.0)
  o_ref[...] = dinv_ref[...] * jnp.dot(h, w_ref[...],
                                       preferred_element_type=jnp.float32)


def _tc_layer(a0, a1, y, dinv, b, w):
  d = y.shape[1]
  n = w.shape[1]
  return pl.pallas_call(
      _layer_body,
      grid=(_GRID,),
      in_specs=[
          pl.BlockSpec((_BN, d), lambda i: (i, 0)),
          pl.BlockSpec((_BN, d), lambda i: (i, 0)),
          pl.BlockSpec((_BN, d), lambda i: (i, 0)),
          pl.BlockSpec((_BN, 1), lambda i: (i, 0)),
          pl.BlockSpec((1, d), lambda i: (0, 0)),
          pl.BlockSpec((d, n), lambda i: (0, 0)),
      ],
      out_specs=pl.BlockSpec((_BN, n), lambda i: (i, 0)),
      out_shape=jax.ShapeDtypeStruct((N, n), jnp.float32),
  )(a0, a1, y, dinv, b, w)


def _combine_body(a0_ref, a1_ref, y_ref, dinv_ref, b_ref, o_ref):
  o_ref[...] = (dinv_ref[...] * (a0_ref[...] + a1_ref[...] + y_ref[...])
                + b_ref[...])


def _tc_combine(a0, a1, y, dinv, b):
  d = y.shape[1]
  return pl.pallas_call(
      _combine_body,
      grid=(_GRID,),
      in_specs=[
          pl.BlockSpec((_BN, d), lambda i: (i, 0)),
          pl.BlockSpec((_BN, d), lambda i: (i, 0)),
          pl.BlockSpec((_BN, d), lambda i: (i, 0)),
          pl.BlockSpec((_BN, 1), lambda i: (i, 0)),
          pl.BlockSpec((1, d), lambda i: (0, 0)),
      ],
      out_specs=pl.BlockSpec((_BN, d), lambda i: (i, 0)),
      out_shape=jax.ShapeDtypeStruct((N, d), jnp.float32),
  )(a0, a1, y, dinv, b)


def kernel(x, edge_index, W1, b1, W2, b2):
  src = edge_index[0]
  dst = edge_index[1]

  ones = jnp.ones((N, 8), jnp.float32)
  d0, d1 = _sc_accumulate(ones, src, dst, 8)
  p0 = d0[:, 0:1]
  p1 = d1[:, 0:1]

  xw1 = _tc_matmul(x, W1)
  dinv, y1 = _tc_scale(p0, p1, xw1)

  a0, a1 = _sc_accumulate(y1, src, dst, 64)
  y2 = _tc_layer(a0, a1, y1, dinv, b1.reshape(1, -1), W2)

  c0, c1 = _sc_accumulate(y2, src, dst, 32)
  return _tc_combine(c0, c1, y2, dinv, b2.reshape(1, -1))


# speedup vs baseline: 33.8708x; 1.1260x over previous
import functools

import jax
import jax.numpy as jnp
from jax import lax
from jax.experimental import pallas as pl
from jax.experimental.pallas import tpu as pltpu
from jax.experimental.pallas import tpu_sc as plsc

N = 10000
E = 320000
NC = 2
NS = 16
NW = NC * NS
EPW = E // NW
K = 125
NCHUNK = EPW // K
RB = 640
RB_LAST = N - 15 * RB


def _make_sc_accumulate(D):
  mesh = plsc.VectorSubcoreMesh(core_axis_name="c", subcore_axis_name="s")

  @functools.partial(
      pl.kernel,
      out_type=jax.ShapeDtypeStruct((NC * N, D), jnp.float32),
      mesh=mesh,
      compiler_params=pltpu.CompilerParams(use_tc_tiling_on_sc=False),
      scratch_types=[
          pltpu.VMEM((NCHUNK, K), jnp.int32),
          pltpu.VMEM((NCHUNK, K), jnp.int32),
          pltpu.VMEM((K, D), jnp.float32),
          pltpu.VMEM((K, D), jnp.float32),
          pltpu.VMEM_SHARED((N, D), jnp.float32),
          pltpu.SemaphoreType.DMA,
          pltpu.SemaphoreType.DMA,
      ],
  )
  def body(y_hbm, src_hbm, dst_hbm, zeros_hbm, out_hbm, src_all, dst_all,
           rows0, rows1, acc_sh, sem0, sem1):
    c = lax.axis_index("c")
    s = lax.axis_index("s")
    wid = c * NS + s
    r0 = s * RB

    @pl.when(s < NS - 1)
    def _():
      pltpu.sync_copy(zeros_hbm.at[pl.ds(r0, RB)], acc_sh.at[pl.ds(r0, RB)])

    @pl.when(s == NS - 1)
    def _():
      pltpu.sync_copy(zeros_hbm.at[pl.ds(r0, RB_LAST)],
                      acc_sh.at[pl.ds(r0, RB_LAST)])

    pltpu.sync_copy(src_hbm.at[wid], src_all)
    pltpu.sync_copy(dst_hbm.at[wid], dst_all)
    plsc.subcore_barrier()

    pltpu.async_copy(y_hbm.at[src_all.at[0]], rows0, sem0)

    def pair(jj, carry):
      j = jj * 2
      pltpu.async_copy(y_hbm.at[src_all.at[j + 1]], rows1, sem1)
      pltpu.make_async_copy(y_hbm.at[src_all.at[j]], rows0, sem0).wait()
      pltpu.sync_copy(rows0, acc_sh.at[dst_all.at[j]], add=True)
      pltpu.async_copy(y_hbm.at[src_all.at[j + 2]], rows0, sem0)
      pltpu.make_async_copy(y_hbm.at[src_all.at[j + 1]], rows1, sem1).wait()
      pltpu.sync_copy(rows1, acc_sh.at[dst_all.at[j + 1]], add=True)
      return carry

    lax.fori_loop(0, NCHUNK // 2 - 1, pair, 0)
    j = NCHUNK - 2
    pltpu.async_copy(y_hbm.at[src_all.at[j + 1]], rows1, sem1)
    pltpu.make_async_copy(y_hbm.at[src_all.at[j]], rows0, sem0).wait()
    pltpu.sync_copy(rows0, acc_sh.at[dst_all.at[j]], add=True)
    pltpu.make_async_copy(y_hbm.at[src_all.at[j + 1]], rows1, sem1).wait()
    pltpu.sync_copy(rows1, acc_sh.at[dst_all.at[j + 1]], add=True)
    plsc.subcore_barrier()

    @pl.when(s < NS - 1)
    def _():
      pltpu.sync_copy(acc_sh.at[pl.ds(r0, RB)],
                      out_hbm.at[pl.ds(c * N + r0, RB)])

    @pl.when(s == NS - 1)
    def _():
      pltpu.sync_copy(acc_sh.at[pl.ds(r0, RB_LAST)],
                      out_hbm.at[pl.ds(c * N + r0, RB_LAST)])

  return body


def _sc_accumulate(y, src, dst, D):
  zeros = jnp.zeros((N, D), jnp.float32)
  srcw = src.reshape(NW, NCHUNK, K)
  dstw = dst.reshape(NW, NCHUNK, K)
  parts = _make_sc_accumulate(D)(y, srcw, dstw, zeros)
  return parts[:N], parts[N:]


_GRID = 10
_BN = N // _GRID


def _scale_body(p0_ref, p1_ref, x_ref, w_ref, dinv_ref, y_ref):
  deg = 1.0 + p0_ref[...] + p1_ref[...]
  dinv = lax.rsqrt(deg)
  dinv_ref[...] = dinv
  xw = jnp.dot(x_ref[...], w_ref[...], preferred_element_type=jnp.float32)
  y_ref[...] = dinv * xw


def _tc_scale_mm(p0, p1, x, w):
  kdim = x.shape[1]
  d = w.shape[1]
  return pl.pallas_call(
      _scale_body,
      grid=(_GRID,),
      in_specs=[
          pl.BlockSpec((_BN, 1), lambda i: (i, 0)),
          pl.BlockSpec((_BN, 1), lambda i: (i, 0)),
          pl.BlockSpec((_BN, kdim), lambda i: (i, 0)),
          pl.BlockSpec((kdim, d), lambda i: (0, 0)),
      ],
      out_specs=[
          pl.BlockSpec((_BN, 1), lambda i: (i, 0)),
          pl.BlockSpec((_BN, d), lambda i: (i, 0)),
      ],
      out_shape=[
          jax.ShapeDtypeStruct((N, 1), jnp.float32),
          jax.ShapeDtypeStruct((N, d), jnp.float32),
      ],
  )(p0, p1, x, w)


def _layer_body(a0_ref, a1_ref, y_ref, dinv_ref, b_ref, w_ref, o_ref):
  h = dinv_ref[...] * (a0_ref[...] + a1_ref[...] + y_ref[...]) + b_ref[...]
  h = jnp.maximum(h, 0.0)
  o_ref[...] = dinv_ref[...] * jnp.dot(h, w_ref[...],
                                       preferred_element_type=jnp.float32)


def _tc_layer(a0, a1, y, dinv, b, w):
  d = y.shape[1]
  n = w.shape[1]
  return pl.pallas_call(
      _layer_body,
      grid=(_GRID,),
      in_specs=[
          pl.BlockSpec((_BN, d), lambda i: (i, 0)),
          pl.BlockSpec((_BN, d), lambda i: (i, 0)),
          pl.BlockSpec((_BN, d), lambda i: (i, 0)),
          pl.BlockSpec((_BN, 1), lambda i: (i, 0)),
          pl.BlockSpec((1, d), lambda i: (0, 0)),
          pl.BlockSpec((d, n), lambda i: (0, 0)),
      ],
      out_specs=pl.BlockSpec((_BN, n), lambda i: (i, 0)),
      out_shape=jax.ShapeDtypeStruct((N, n), jnp.float32),
  )(a0, a1, y, dinv, b, w)


def _combine_body(a0_ref, a1_ref, y_ref, dinv_ref, b_ref, o_ref):
  o_ref[...] = (dinv_ref[...] * (a0_ref[...] + a1_ref[...] + y_ref[...])
                + b_ref[...])


def _tc_combine(a0, a1, y, dinv, b):
  d = y.shape[1]
  return pl.pallas_call(
      _combine_body,
      grid=(_GRID,),
      in_specs=[
          pl.BlockSpec((_BN, d), lambda i: (i, 0)),
          pl.BlockSpec((_BN, d), lambda i: (i, 0)),
          pl.BlockSpec((_BN, d), lambda i: (i, 0)),
          pl.BlockSpec((_BN, 1), lambda i: (i, 0)),
          pl.BlockSpec((1, d), lambda i: (0, 0)),
      ],
      out_specs=pl.BlockSpec((_BN, d), lambda i: (i, 0)),
      out_shape=jax.ShapeDtypeStruct((N, d), jnp.float32),
  )(a0, a1, y, dinv, b)


def kernel(x, edge_index, W1, b1, W2, b2):
  src = edge_index[0]
  dst = edge_index[1]

  ones = jnp.ones((N, 8), jnp.float32)
  d0, d1 = _sc_accumulate(ones, src, dst, 8)
  p0 = d0[:, 0:1]
  p1 = d1[:, 0:1]

  dinv, y1 = _tc_scale_mm(p0, p1, x, W1)

  a0, a1 = _sc_accumulate(y1, src, dst, 64)
  y2 = _tc_layer(a0, a1, y1, dinv, b1.reshape(1, -1), W2)

  c0, c1 = _sc_accumulate(y2, src, dst, 32)
  return _tc_combine(c0, c1, y2, dinv, b2.reshape(1, -1))


# speedup vs baseline: 39.2826x; 1.1598x over previous
import functools

import jax
import jax.numpy as jnp
from jax import lax
from jax.experimental import pallas as pl
from jax.experimental.pallas import tpu as pltpu
from jax.experimental.pallas import tpu_sc as plsc

N = 10000
E = 320000
NC = 2
NS = 16
NW = NC * NS
EPW = E // NW
K = 125
NCHUNK = EPW // K
RB = 640
RB_LAST = N - 15 * RB


def _make_sc_accumulate(D):
  mesh = plsc.VectorSubcoreMesh(core_axis_name="c", subcore_axis_name="s")

  @functools.partial(
      pl.kernel,
      out_type=jax.ShapeDtypeStruct((NC * N, D), jnp.float32),
      mesh=mesh,
      compiler_params=pltpu.CompilerParams(use_tc_tiling_on_sc=False),
      scratch_types=[
          pltpu.VMEM((NCHUNK, K), jnp.int32),
          pltpu.VMEM((NCHUNK, K), jnp.int32),
          pltpu.VMEM((4, K, D), jnp.float32),
          pltpu.VMEM_SHARED((N, D), jnp.float32),
          [pltpu.SemaphoreType.DMA] * 4,
          [pltpu.SemaphoreType.DMA] * 4,
      ],
  )
  def body(y_hbm, src_hbm, dst_hbm, zeros_hbm, out_hbm, src_all, dst_all,
           ring, acc_sh, gsems, ssems):
    c = lax.axis_index("c")
    s = lax.axis_index("s")
    wid = c * NS + s
    r0 = s * RB

    @pl.when(s < NS - 1)
    def _():
      pltpu.sync_copy(zeros_hbm.at[pl.ds(r0, RB)], acc_sh.at[pl.ds(r0, RB)])

    @pl.when(s == NS - 1)
    def _():
      pltpu.sync_copy(zeros_hbm.at[pl.ds(r0, RB_LAST)],
                      acc_sh.at[pl.ds(r0, RB_LAST)])

    pltpu.sync_copy(src_hbm.at[wid], src_all)
    pltpu.sync_copy(dst_hbm.at[wid], dst_all)
    plsc.subcore_barrier()

    def g_issue(i, t):
      pltpu.async_copy(y_hbm.at[src_all.at[i]], ring.at[t], gsems[t])

    def g_wait(i, t):
      pltpu.make_async_copy(y_hbm.at[src_all.at[i]], ring.at[t],
                            gsems[t]).wait()

    def s_issue(i, t):
      pltpu.async_copy(ring.at[t], acc_sh.at[dst_all.at[i]], ssems[t],
                       add=True)

    def s_wait(i, t):
      pltpu.make_async_copy(ring.at[t], acc_sh.at[dst_all.at[i]],
                            ssems[t]).wait()

    g_issue(0, 0)
    g_issue(1, 1)
    g_issue(2, 2)
    g_wait(0, 0)
    s_issue(0, 0)
    g_issue(3, 3)
    g_wait(1, 1)
    s_issue(1, 1)
    s_wait(0, 0)
    g_issue(4, 0)
    g_wait(2, 2)
    s_issue(2, 2)
    s_wait(1, 1)
    g_issue(5, 1)
    g_wait(3, 3)
    s_issue(3, 3)

    def quad(jj, carry):
      base = jj * 4
      for t in range(4):
        i = base + t
        s_wait(i - 2, (t + 2) % 4)
        g_issue(i + 2, (t + 2) % 4)
        g_wait(i, t)
        s_issue(i, t)
      return carry

    lax.fori_loop(1, NCHUNK // 4 - 1, quad, 0)

    e = NCHUNK - 4
    s_wait(e - 2, 2)
    g_issue(e + 2, 2)
    g_wait(e, 0)
    s_issue(e, 0)
    s_wait(e - 1, 3)
    g_issue(e + 3, 3)
    g_wait(e + 1, 1)
    s_issue(e + 1, 1)
    g_wait(e + 2, 2)
    s_issue(e + 2, 2)
    g_wait(e + 3, 3)
    s_issue(e + 3, 3)
    s_wait(e, 0)
    s_wait(e + 1, 1)
    s_wait(e + 2, 2)
    s_wait(e + 3, 3)
    plsc.subcore_barrier()

    @pl.when(s < NS - 1)
    def _():
      pltpu.sync_copy(acc_sh.at[pl.ds(r0, RB)],
                      out_hbm.at[pl.ds(c * N + r0, RB)])

    @pl.when(s == NS - 1)
    def _():
      pltpu.sync_copy(acc_sh.at[pl.ds(r0, RB_LAST)],
                      out_hbm.at[pl.ds(c * N + r0, RB_LAST)])

  return body


def _sc_accumulate(y, src, dst, D):
  zeros = jnp.zeros((N, D), jnp.float32)
  srcw = src.reshape(NW, NCHUNK, K)
  dstw = dst.reshape(NW, NCHUNK, K)
  parts = _make_sc_accumulate(D)(y, srcw, dstw, zeros)
  return parts[:N], parts[N:]


_GRID = 10
_BN = N // _GRID


def _scale_body(p0_ref, p1_ref, x_ref, w_ref, dinv_ref, y_ref):
  deg = 1.0 + p0_ref[...] + p1_ref[...]
  dinv = lax.rsqrt(deg)
  dinv_ref[...] = dinv
  xw = jnp.dot(x_ref[...], w_ref[...], preferred_element_type=jnp.float32)
  y_ref[...] = dinv * xw


def _tc_scale_mm(p0, p1, x, w):
  kdim = x.shape[1]
  d = w.shape[1]
  return pl.pallas_call(
      _scale_body,
      grid=(_GRID,),
      in_specs=[
          pl.BlockSpec((_BN, 1), lambda i: (i, 0)),
          pl.BlockSpec((_BN, 1), lambda i: (i, 0)),
          pl.BlockSpec((_BN, kdim), lambda i: (i, 0)),
          pl.BlockSpec((kdim, d), lambda i: (0, 0)),
      ],
      out_specs=[
          pl.BlockSpec((_BN, 1), lambda i: (i, 0)),
          pl.BlockSpec((_BN, d), lambda i: (i, 0)),
      ],
      out_shape=[
          jax.ShapeDtypeStruct((N, 1), jnp.float32),
          jax.ShapeDtypeStruct((N, d), jnp.float32),
      ],
  )(p0, p1, x, w)


def _layer_body(a0_ref, a1_ref, y_ref, dinv_ref, b_ref, w_ref, o_ref):
  h = dinv_ref[...] * (a0_ref[...] + a1_ref[...] + y_ref[...]) + b_ref[...]
  h = jnp.maximum(h, 0.0)
  o_ref[...] = dinv_ref[...] * jnp.dot(h, w_ref[...],
                                       preferred_element_type=jnp.float32)


def _tc_layer(a0, a1, y, dinv, b, w):
  d = y.shape[1]
  n = w.shape[1]
  return pl.pallas_call(
      _layer_body,
      grid=(_GRID,),
      in_specs=[
          pl.BlockSpec((_BN, d), lambda i: (i, 0)),
          pl.BlockSpec((_BN, d), lambda i: (i, 0)),
          pl.BlockSpec((_BN, d), lambda i: (i, 0)),
          pl.BlockSpec((_BN, 1), lambda i: (i, 0)),
          pl.BlockSpec((1, d), lambda i: (0, 0)),
          pl.BlockSpec((d, n), lambda i: (0, 0)),
      ],
      out_specs=pl.BlockSpec((_BN, n), lambda i: (i, 0)),
      out_shape=jax.ShapeDtypeStruct((N, n), jnp.float32),
  )(a0, a1, y, dinv, b, w)


def _combine_body(a0_ref, a1_ref, y_ref, dinv_ref, b_ref, o_ref):
  o_ref[...] = (dinv_ref[...] * (a0_ref[...] + a1_ref[...] + y_ref[...])
                + b_ref[...])


def _tc_combine(a0, a1, y, dinv, b):
  d = y.shape[1]
  return pl.pallas_call(
      _combine_body,
      grid=(_GRID,),
      in_specs=[
          pl.BlockSpec((_BN, d), lambda i: (i, 0)),
          pl.BlockSpec((_BN, d), lambda i: (i, 0)),
          pl.BlockSpec((_BN, d), lambda i: (i, 0)),
          pl.BlockSpec((_BN, 1), lambda i: (i, 0)),
          pl.BlockSpec((1, d), lambda i: (0, 0)),
      ],
      out_specs=pl.BlockSpec((_BN, d), lambda i: (i, 0)),
      out_shape=jax.ShapeDtypeStruct((N, d), jnp.float32),
  )(a0, a1, y, dinv, b)


def kernel(x, edge_index, W1, b1, W2, b2):
  src = edge_index[0]
  dst = edge_index[1]

  ones = jnp.ones((N, 8), jnp.float32)
  d0, d1 = _sc_accumulate(ones, src, dst, 8)
  p0 = d0[:, 0:1]
  p1 = d1[:, 0:1]

  dinv, y1 = _tc_scale_mm(p0, p1, x, W1)

  a0, a1 = _sc_accumulate(y1, src, dst, 64)
  y2 = _tc_layer(a0, a1, y1, dinv, b1.reshape(1, -1), W2)

  c0, c1 = _sc_accumulate(y2, src, dst, 32)
  return _tc_combine(c0, c1, y2, dinv, b2.reshape(1, -1))


# speedup vs baseline: 47.1053x; 1.1991x over previous
import functools

import jax
import jax.numpy as jnp
from jax import lax
from jax.experimental import pallas as pl
from jax.experimental.pallas import tpu as pltpu
from jax.experimental.pallas import tpu_sc as plsc

N = 10000
E = 320000
NC = 2
NS = 16
NW = NC * NS
EPW = E // NW
K = 125
NCHUNK = EPW // K
ZR = 80
RB = 640
RB_LAST = N - 15 * RB

_MESH = plsc.VectorSubcoreMesh(core_axis_name="c", subcore_axis_name="s")
_SC_PARAMS = pltpu.CompilerParams(use_tc_tiling_on_sc=False)


def _zero_band(s, zbuf, acc_sh, zsem):
  r0 = s * RB

  @pl.when(s < NS - 1)
  def _():
    for t in range(RB // ZR):
      pltpu.async_copy(zbuf, acc_sh.at[pl.ds(r0 + t * ZR, ZR)], zsem)
    for t in range(RB // ZR):
      pltpu.make_async_copy(zbuf, acc_sh.at[pl.ds(r0 + t * ZR, ZR)],
                            zsem).wait()

  @pl.when(s == NS - 1)
  def _():
    for t in range(RB_LAST // ZR):
      pltpu.async_copy(zbuf, acc_sh.at[pl.ds(r0 + t * ZR, ZR)], zsem)
    for t in range(RB_LAST // ZR):
      pltpu.make_async_copy(zbuf, acc_sh.at[pl.ds(r0 + t * ZR, ZR)],
                            zsem).wait()


def _drain_band(s, c, acc_sh, out_hbm):
  r0 = s * RB

  @pl.when(s < NS - 1)
  def _():
    pltpu.sync_copy(acc_sh.at[pl.ds(r0, RB)],
                    out_hbm.at[pl.ds(c * N + r0, RB)])

  @pl.when(s == NS - 1)
  def _():
    pltpu.sync_copy(acc_sh.at[pl.ds(r0, RB_LAST)],
                    out_hbm.at[pl.ds(c * N + r0, RB_LAST)])


def _make_sc_accumulate(D):
  @functools.partial(
      pl.kernel,
      out_type=jax.ShapeDtypeStruct((NC * N, D), jnp.float32),
      mesh=_MESH,
      compiler_params=_SC_PARAMS,
      scratch_types=[
          pltpu.VMEM((NCHUNK, K), jnp.int32),
          pltpu.VMEM((NCHUNK, K), jnp.int32),
          pltpu.VMEM((4, K, D), jnp.float32),
          pltpu.VMEM((ZR, D), jnp.float32),
          pltpu.VMEM_SHARED((N, D), jnp.float32),
          [pltpu.SemaphoreType.DMA] * 4,
          [pltpu.SemaphoreType.DMA] * 4,
      ],
  )
  def body(y_hbm, src_hbm, dst_hbm, ztile_hbm, out_hbm, src_all, dst_all,
           ring, zbuf, acc_sh, gsems, ssems):
    c = lax.axis_index("c")
    s = lax.axis_index("s")
    wid = c * NS + s

    pltpu.sync_copy(src_hbm.at[wid], src_all)
    pltpu.sync_copy(dst_hbm.at[wid], dst_all)
    pltpu.sync_copy(ztile_hbm, zbuf)
    _zero_band(s, zbuf, acc_sh, gsems[0])
    plsc.subcore_barrier()

    def g_issue(i, t):
      pltpu.async_copy(y_hbm.at[src_all.at[i]], ring.at[t], gsems[t])

    def g_wait(i, t):
      pltpu.make_async_copy(y_hbm.at[src_all.at[i]], ring.at[t],
                            gsems[t]).wait()

    def s_issue(i, t):
      pltpu.async_copy(ring.at[t], acc_sh.at[dst_all.at[i]], ssems[t],
                       add=True)

    def s_wait(i, t):
      pltpu.make_async_copy(ring.at[t], acc_sh.at[dst_all.at[i]],
                            ssems[t]).wait()

    g_issue(0, 0)
    g_issue(1, 1)
    g_issue(2, 2)
    g_wait(0, 0)
    s_issue(0, 0)
    g_issue(3, 3)
    g_wait(1, 1)
    s_issue(1, 1)
    s_wait(0, 0)
    g_issue(4, 0)
    g_wait(2, 2)
    s_issue(2, 2)
    s_wait(1, 1)
    g_issue(5, 1)
    g_wait(3, 3)
    s_issue(3, 3)

    def quad(jj, carry):
      base = jj * 4
      for t in range(4):
        i = base + t
        s_wait(i - 2, (t + 2) % 4)
        g_issue(i + 2, (t + 2) % 4)
        g_wait(i, t)
        s_issue(i, t)
      return carry

    lax.fori_loop(1, NCHUNK // 4 - 1, quad, 0)

    e = NCHUNK - 4
    s_wait(e - 2, 2)
    g_issue(e + 2, 2)
    g_wait(e, 0)
    s_issue(e, 0)
    s_wait(e - 1, 3)
    g_issue(e + 3, 3)
    g_wait(e + 1, 1)
    s_issue(e + 1, 1)
    g_wait(e + 2, 2)
    s_issue(e + 2, 2)
    g_wait(e + 3, 3)
    s_issue(e + 3, 3)
    s_wait(e, 0)
    s_wait(e + 1, 1)
    s_wait(e + 2, 2)
    s_wait(e + 3, 3)
    plsc.subcore_barrier()

    _drain_band(s, c, acc_sh, out_hbm)

  return body


@functools.partial(
    pl.kernel,
    out_type=jax.ShapeDtypeStruct((NC * N, 8), jnp.float32),
    mesh=_MESH,
    compiler_params=_SC_PARAMS,
    scratch_types=[
        pltpu.VMEM((NCHUNK, K), jnp.int32),
        pltpu.VMEM((K, 8), jnp.float32),
        pltpu.VMEM((ZR, 8), jnp.float32),
        pltpu.VMEM_SHARED((N, 8), jnp.float32),
        [pltpu.SemaphoreType.DMA] * 4,
    ],
)
def _sc_degree_body(dst_hbm, ones_hbm, ztile_hbm, out_hbm, dst_all, ones_v,
                    zbuf, acc_sh, ssems):
  c = lax.axis_index("c")
  s = lax.axis_index("s")
  wid = c * NS + s

  pltpu.sync_copy(dst_hbm.at[wid], dst_all)
  pltpu.sync_copy(ones_hbm, ones_v)
  pltpu.sync_copy(ztile_hbm, zbuf)
  _zero_band(s, zbuf, acc_sh, ssems[0])
  plsc.subcore_barrier()

  def s_issue(i, t):
    pltpu.async_copy(ones_v, acc_sh.at[dst_all.at[i]], ssems[t], add=True)

  def s_wait(i, t):
    pltpu.make_async_copy(ones_v, acc_sh.at[dst_all.at[i]], ssems[t]).wait()

  s_issue(0, 0)
  s_issue(1, 1)
  s_issue(2, 2)
  s_issue(3, 3)

  def quad(jj, carry):
    base = jj * 4
    for t in range(4):
      i = base + t
      s_wait(i - 4, t)
      s_issue(i, t)
    return carry

  lax.fori_loop(1, NCHUNK // 4, quad, 0)
  for t in range(4):
    s_wait(NCHUNK - 4 + t, t)
  plsc.subcore_barrier()

  _drain_band(s, c, acc_sh, out_hbm)


_GRID = 10
_BN = N // _GRID


def _scale_body(p0_ref, p1_ref, x_ref, w_ref, dinv_ref, y_ref):
  deg = 1.0 + p0_ref[:, 0:1] + p1_ref[:, 0:1]
  dinv = lax.rsqrt(deg)
  dinv_ref[...] = dinv
  xw = jnp.dot(x_ref[...], w_ref[...], preferred_element_type=jnp.float32)
  y_ref[...] = dinv * xw


def _tc_scale_mm(degparts, x, w):
  kdim = x.shape[1]
  d = w.shape[1]
  return pl.pallas_call(
      _scale_body,
      grid=(_GRID,),
      in_specs=[
          pl.BlockSpec((_BN, 8), lambda i: (i, 0)),
          pl.BlockSpec((_BN, 8), lambda i: (i + _GRID, 0)),
          pl.BlockSpec((_BN, kdim), lambda i: (i, 0)),
          pl.BlockSpec((kdim, d), lambda i: (0, 0)),
      ],
      out_specs=[
          pl.BlockSpec((_BN, 1), lambda i: (i, 0)),
          pl.BlockSpec((_BN, d), lambda i: (i, 0)),
      ],
      out_shape=[
          jax.ShapeDtypeStruct((N, 1), jnp.float32),
          jax.ShapeDtypeStruct((N, d), jnp.float32),
      ],
  )(degparts, degparts, x, w)


def _layer_body(a0_ref, a1_ref, y_ref, dinv_ref, b_ref, w_ref, o_ref):
  h = dinv_ref[...] * (a0_ref[...] + a1_ref[...] + y_ref[...]) + b_ref[...]
  h = jnp.maximum(h, 0.0)
  o_ref[...] = dinv_ref[...] * jnp.dot(h, w_ref[...],
                                       preferred_element_type=jnp.float32)


def _tc_layer(accparts, y, dinv, b, w):
  d = y.shape[1]
  n = w.shape[1]
  return pl.pallas_call(
      _layer_body,
      grid=(_GRID,),
      in_specs=[
          pl.BlockSpec((_BN, d), lambda i: (i, 0)),
          pl.BlockSpec((_BN, d), lambda i: (i + _GRID, 0)),
          pl.BlockSpec((_BN, d), lambda i: (i, 0)),
          pl.BlockSpec((_BN, 1), lambda i: (i, 0)),
          pl.BlockSpec((1, d), lambda i: (0, 0)),
          pl.BlockSpec((d, n), lambda i: (0, 0)),
      ],
      out_specs=pl.BlockSpec((_BN, n), lambda i: (i, 0)),
      out_shape=jax.ShapeDtypeStruct((N, n), jnp.float32),
  )(accparts, accparts, y, dinv, b, w)


def _combine_body(a0_ref, a1_ref, y_ref, dinv_ref, b_ref, o_ref):
  o_ref[...] = (dinv_ref[...] * (a0_ref[...] + a1_ref[...] + y_ref[...])
                + b_ref[...])


def _tc_combine(accparts, y, dinv, b):
  d = y.shape[1]
  return pl.pallas_call(
      _combine_body,
      grid=(_GRID,),
      in_specs=[
          pl.BlockSpec((_BN, d), lambda i: (i, 0)),
          pl.BlockSpec((_BN, d), lambda i: (i + _GRID, 0)),
          pl.BlockSpec((_BN, d), lambda i: (i, 0)),
          pl.BlockSpec((_BN, 1), lambda i: (i, 0)),
          pl.BlockSpec((1, d), lambda i: (0, 0)),
      ],
      out_specs=pl.BlockSpec((_BN, d), lambda i: (i, 0)),
      out_shape=jax.ShapeDtypeStruct((N, d), jnp.float32),
  )(accparts, accparts, y, dinv, b)


def kernel(x, edge_index, W1, b1, W2, b2):
  srcw = edge_index[0].reshape(NW, NCHUNK, K)
  dstw = edge_index[1].reshape(NW, NCHUNK, K)

  degparts = _sc_degree_body(dstw, jnp.ones((K, 8), jnp.float32),
                             jnp.zeros((ZR, 8), jnp.float32))
  dinv, y1 = _tc_scale_mm(degparts, x, W1)

  acc1 = _make_sc_accumulate(64)(y1, srcw, dstw,
                                 jnp.zeros((ZR, 64), jnp.float32))
  y2 = _tc_layer(acc1, y1, dinv, b1.reshape(1, -1), W2)

  acc2 = _make_sc_accumulate(32)(y2, srcw, dstw,
                                 jnp.zeros((ZR, 32), jnp.float32))
  return _tc_combine(acc2, y2, dinv, b2.reshape(1, -1))


# speedup vs baseline: 53.6377x; 1.1387x over previous
import functools

import jax
import jax.numpy as jnp
from jax import lax
from jax.experimental import pallas as pl
from jax.experimental.pallas import tpu as pltpu
from jax.experimental.pallas import tpu_sc as plsc

N = 10000
E = 320000
NC = 2
NS = 16
NW = NC * NS
EPW = E // NW
K = 125
NCHUNK = EPW // K
ZR = 80
RB = 640
RB_LAST = N - 15 * RB

_MESH = plsc.VectorSubcoreMesh(core_axis_name="c", subcore_axis_name="s")
_SC_PARAMS = pltpu.CompilerParams(use_tc_tiling_on_sc=False)


def _zero_band(s, zbuf, acc_sh, zsem):
  r0 = s * RB

  @pl.when(s < NS - 1)
  def _():
    for t in range(RB // ZR):
      pltpu.async_copy(zbuf, acc_sh.at[pl.ds(r0 + t * ZR, ZR)], zsem)
    for t in range(RB // ZR):
      pltpu.make_async_copy(zbuf, acc_sh.at[pl.ds(r0 + t * ZR, ZR)],
                            zsem).wait()

  @pl.when(s == NS - 1)
  def _():
    for t in range(RB_LAST // ZR):
      pltpu.async_copy(zbuf, acc_sh.at[pl.ds(r0 + t * ZR, ZR)], zsem)
    for t in range(RB_LAST // ZR):
      pltpu.make_async_copy(zbuf, acc_sh.at[pl.ds(r0 + t * ZR, ZR)],
                            zsem).wait()


def _drain_band(s, c, acc_sh, out_hbm, D):
  r0 = s * RB

  @pl.when(s < NS - 1)
  def _():
    pltpu.sync_copy(acc_sh.at[pl.ds(r0, RB)],
                    out_hbm.at[pl.ds(c * N + r0, RB), pl.ds(0, D)])

  @pl.when(s == NS - 1)
  def _():
    pltpu.sync_copy(acc_sh.at[pl.ds(r0, RB_LAST)],
                    out_hbm.at[pl.ds(c * N + r0, RB_LAST), pl.ds(0, D)])


def _make_sc_accumulate(D):
  @functools.partial(
      pl.kernel,
      out_type=jax.ShapeDtypeStruct((NC * N, 128), jnp.float32),
      mesh=_MESH,
      compiler_params=_SC_PARAMS,
      scratch_types=[
          pltpu.VMEM((NCHUNK, K), jnp.int32),
          pltpu.VMEM((NCHUNK, K), jnp.int32),
          pltpu.VMEM((4, K, D), jnp.float32),
          pltpu.VMEM((ZR, D), jnp.float32),
          pltpu.VMEM_SHARED((N, D), jnp.float32),
          [pltpu.SemaphoreType.DMA] * 4,
          [pltpu.SemaphoreType.DMA] * 4,
      ],
  )
  def body(y_hbm, src_hbm, dst_hbm, ztile_hbm, out_hbm, src_all, dst_all,
           ring, zbuf, acc_sh, gsems, ssems):
    c = lax.axis_index("c")
    s = lax.axis_index("s")
    wid = c * NS + s

    pltpu.sync_copy(src_hbm.at[wid], src_all)
    pltpu.sync_copy(dst_hbm.at[wid], dst_all)
    pltpu.sync_copy(ztile_hbm, zbuf)
    _zero_band(s, zbuf, acc_sh, gsems[0])
    plsc.subcore_barrier()

    def g_issue(i, t):
      pltpu.async_copy(y_hbm.at[src_all.at[i]], ring.at[t], gsems[t])

    def g_wait(i, t):
      pltpu.make_async_copy(y_hbm.at[src_all.at[i]], ring.at[t],
                            gsems[t]).wait()

    def s_issue(i, t):
      pltpu.async_copy(ring.at[t], acc_sh.at[dst_all.at[i]], ssems[t],
                       add=True)

    def s_wait(i, t):
      pltpu.make_async_copy(ring.at[t], acc_sh.at[dst_all.at[i]],
                            ssems[t]).wait()

    g_issue(0, 0)
    g_issue(1, 1)
    g_issue(2, 2)
    g_wait(0, 0)
    s_issue(0, 0)
    g_issue(3, 3)
    g_wait(1, 1)
    s_issue(1, 1)
    s_wait(0, 0)
    g_issue(4, 0)
    g_wait(2, 2)
    s_issue(2, 2)
    s_wait(1, 1)
    g_issue(5, 1)
    g_wait(3, 3)
    s_issue(3, 3)

    def quad(jj, carry):
      base = jj * 4
      for t in range(4):
        i = base + t
        s_wait(i - 2, (t + 2) % 4)
        g_issue(i + 2, (t + 2) % 4)
        g_wait(i, t)
        s_issue(i, t)
      return carry

    lax.fori_loop(1, NCHUNK // 4 - 1, quad, 0)

    e = NCHUNK - 4
    s_wait(e - 2, 2)
    g_issue(e + 2, 2)
    g_wait(e, 0)
    s_issue(e, 0)
    s_wait(e - 1, 3)
    g_issue(e + 3, 3)
    g_wait(e + 1, 1)
    s_issue(e + 1, 1)
    g_wait(e + 2, 2)
    s_issue(e + 2, 2)
    g_wait(e + 3, 3)
    s_issue(e + 3, 3)
    s_wait(e, 0)
    s_wait(e + 1, 1)
    s_wait(e + 2, 2)
    s_wait(e + 3, 3)
    plsc.subcore_barrier()

    _drain_band(s, c, acc_sh, out_hbm, D)

  return body


@functools.partial(
    pl.kernel,
    out_type=jax.ShapeDtypeStruct((NC * N, 128), jnp.float32),
    mesh=_MESH,
    compiler_params=_SC_PARAMS,
    scratch_types=[
        pltpu.VMEM((NCHUNK, K), jnp.int32),
        pltpu.VMEM((K, 8), jnp.float32),
        pltpu.VMEM((ZR, 8), jnp.float32),
        pltpu.VMEM_SHARED((N, 8), jnp.float32),
        [pltpu.SemaphoreType.DMA] * 4,
    ],
)
def _sc_degree_body(dst_hbm, ones_hbm, ztile_hbm, out_hbm, dst_all, ones_v,
                    zbuf, acc_sh, ssems):
  c = lax.axis_index("c")
  s = lax.axis_index("s")
  wid = c * NS + s

  pltpu.sync_copy(dst_hbm.at[wid], dst_all)
  pltpu.sync_copy(ones_hbm, ones_v)
  pltpu.sync_copy(ztile_hbm, zbuf)
  _zero_band(s, zbuf, acc_sh, ssems[0])
  plsc.subcore_barrier()

  def s_issue(i, t):
    pltpu.async_copy(ones_v, acc_sh.at[dst_all.at[i]], ssems[t], add=True)

  def s_wait(i, t):
    pltpu.make_async_copy(ones_v, acc_sh.at[dst_all.at[i]], ssems[t]).wait()

  s_issue(0, 0)
  s_issue(1, 1)
  s_issue(2, 2)
  s_issue(3, 3)

  def quad(jj, carry):
    base = jj * 4
    for t in range(4):
      i = base + t
      s_wait(i - 4, t)
      s_issue(i, t)
    return carry

  lax.fori_loop(1, NCHUNK // 4, quad, 0)
  for t in range(4):
    s_wait(NCHUNK - 4 + t, t)
  plsc.subcore_barrier()

  _drain_band(s, c, acc_sh, out_hbm, 8)


def _scale_body(p_ref, x_ref, w_ref, dinv_ref, y_ref):
  dinv = lax.rsqrt(1.0 + p_ref[0:N, 0:1] + p_ref[N:NC * N, 0:1])
  dinv_ref[...] = dinv
  xw = jnp.dot(x_ref[...], w_ref[...], preferred_element_type=jnp.float32)
  y_ref[...] = dinv * xw


def _tc_scale_mm(degparts, x, w):
  return pl.pallas_call(
      _scale_body,
      out_shape=[
          jax.ShapeDtypeStruct((N, 1), jnp.float32),
          jax.ShapeDtypeStruct((N, w.shape[1]), jnp.float32),
      ],
  )(degparts, x, w)


def _layer_body(a_ref, y_ref, dinv_ref, b_ref, w_ref, o_ref):
  d = w_ref.shape[0]
  t = a_ref[0:N, 0:d] + a_ref[N:NC * N, 0:d] + y_ref[...]
  h = jnp.maximum(dinv_ref[...] * t + b_ref[...], 0.0)
  o_ref[...] = dinv_ref[...] * jnp.dot(h, w_ref[...],
                                       preferred_element_type=jnp.float32)


def _tc_layer(accparts, y, dinv, b, w):
  return pl.pallas_call(
      _layer_body,
      out_shape=jax.ShapeDtypeStruct((N, w.shape[1]), jnp.float32),
  )(accparts, y, dinv, b, w)


def _combine_body(a_ref, y_ref, dinv_ref, b_ref, o_ref):
  d = b_ref.shape[1]
  t = a_ref[0:N, 0:d] + a_ref[N:NC * N, 0:d] + y_ref[...]
  o_ref[...] = dinv_ref[...] * t + b_ref[...]


def _tc_combine(accparts, y, dinv, b):
  d = b.shape[1]
  return pl.pallas_call(
      _combine_body,
      out_shape=jax.ShapeDtypeStruct((N, d), jnp.float32),
  )(accparts, y, dinv, b)


def kernel(x, edge_index, W1, b1, W2, b2):
  srcw = edge_index[0].reshape(NW, NCHUNK, K)
  dstw = edge_index[1].reshape(NW, NCHUNK, K)

  degparts = _sc_degree_body(dstw, jnp.ones((K, 8), jnp.float32),
                             jnp.zeros((ZR, 8), jnp.float32))
  dinv, y1 = _tc_scale_mm(degparts, x, W1)

  acc1 = _make_sc_accumulate(64)(y1, srcw, dstw,
                                 jnp.zeros((ZR, 64), jnp.float32))
  y2 = _tc_layer(acc1, y1, dinv, b1.reshape(1, -1), W2)

  acc2 = _make_sc_accumulate(32)(y2, srcw, dstw,
                                 jnp.zeros((ZR, 32), jnp.float32))
  return _tc_combine(acc2, y2, dinv, b2.reshape(1, -1))


# speedup vs baseline: 54.1374x; 1.0093x over previous
import functools

import jax
import jax.numpy as jnp
from jax import lax
from jax.experimental import pallas as pl
from jax.experimental.pallas import tpu as pltpu
from jax.experimental.pallas import tpu_sc as plsc

N = 10000
E = 320000
NC = 2
NS = 16
NW = NC * NS
EPW = E // NW
K = 125
NCHUNK = EPW // K
ZR = 80
RB = 640
RB_LAST = N - 15 * RB

_MESH = plsc.VectorSubcoreMesh(core_axis_name="c", subcore_axis_name="s")
_SC_PARAMS = pltpu.CompilerParams(use_tc_tiling_on_sc=False)


def _zero_band(s, zbuf, acc_sh, zsem):
  r0 = s * RB

  @pl.when(s < NS - 1)
  def _():
    for t in range(RB // ZR):
      pltpu.async_copy(zbuf, acc_sh.at[pl.ds(r0 + t * ZR, ZR)], zsem)
    for t in range(RB // ZR):
      pltpu.make_async_copy(zbuf, acc_sh.at[pl.ds(r0 + t * ZR, ZR)],
                            zsem).wait()

  @pl.when(s == NS - 1)
  def _():
    for t in range(RB_LAST // ZR):
      pltpu.async_copy(zbuf, acc_sh.at[pl.ds(r0 + t * ZR, ZR)], zsem)
    for t in range(RB_LAST // ZR):
      pltpu.make_async_copy(zbuf, acc_sh.at[pl.ds(r0 + t * ZR, ZR)],
                            zsem).wait()


def _drain_band(s, c, acc_sh, out_hbm, D, stride):
  r0 = s * RB

  @pl.when(s < NS - 1)
  def _():
    pltpu.sync_copy(acc_sh.at[pl.ds(r0, RB)],
                    out_hbm.at[pl.ds(r0, RB), pl.ds(c * stride, D)])

  @pl.when(s == NS - 1)
  def _():
    pltpu.sync_copy(acc_sh.at[pl.ds(r0, RB_LAST)],
                    out_hbm.at[pl.ds(r0, RB_LAST), pl.ds(c * stride, D)])


def _make_sc_accumulate(D):
  @functools.partial(
      pl.kernel,
      out_type=jax.ShapeDtypeStruct((N, 128), jnp.float32),
      mesh=_MESH,
      compiler_params=_SC_PARAMS,
      scratch_types=[
          pltpu.VMEM((NCHUNK, K), jnp.int32),
          pltpu.VMEM((NCHUNK, K), jnp.int32),
          pltpu.VMEM((4, K, D), jnp.float32),
          pltpu.VMEM((ZR, D), jnp.float32),
          pltpu.VMEM_SHARED((N, D), jnp.float32),
          [pltpu.SemaphoreType.DMA] * 4,
          [pltpu.SemaphoreType.DMA] * 4,
      ],
  )
  def body(y_hbm, src_hbm, dst_hbm, ztile_hbm, out_hbm, src_all, dst_all,
           ring, zbuf, acc_sh, gsems, ssems):
    c = lax.axis_index("c")
    s = lax.axis_index("s")
    wid = c * NS + s

    pltpu.sync_copy(src_hbm.at[wid], src_all)
    pltpu.sync_copy(dst_hbm.at[wid], dst_all)
    pltpu.sync_copy(ztile_hbm, zbuf)
    _zero_band(s, zbuf, acc_sh, gsems[0])
    plsc.subcore_barrier()

    def g_issue(i, t):
      pltpu.async_copy(y_hbm.at[src_all.at[i]], ring.at[t], gsems[t])

    def g_wait(i, t):
      pltpu.make_async_copy(y_hbm.at[src_all.at[i]], ring.at[t],
                            gsems[t]).wait()

    def s_issue(i, t):
      pltpu.async_copy(ring.at[t], acc_sh.at[dst_all.at[i]], ssems[t],
                       add=True)

    def s_wait(i, t):
      pltpu.make_async_copy(ring.at[t], acc_sh.at[dst_all.at[i]],
                            ssems[t]).wait()

    g_issue(0, 0)
    g_issue(1, 1)
    g_issue(2, 2)
    g_wait(0, 0)
    s_issue(0, 0)
    g_issue(3, 3)
    g_wait(1, 1)
    s_issue(1, 1)
    s_wait(0, 0)
    g_issue(4, 0)
    g_wait(2, 2)
    s_issue(2, 2)
    s_wait(1, 1)
    g_issue(5, 1)
    g_wait(3, 3)
    s_issue(3, 3)

    def quad(jj, carry):
      base = jj * 4
      for t in range(4):
        i = base + t
        s_wait(i - 2, (t + 2) % 4)
        g_issue(i + 2, (t + 2) % 4)
        g_wait(i, t)
        s_issue(i, t)
      return carry

    lax.fori_loop(1, NCHUNK // 4 - 1, quad, 0)

    e = NCHUNK - 4
    s_wait(e - 2, 2)
    g_issue(e + 2, 2)
    g_wait(e, 0)
    s_issue(e, 0)
    s_wait(e - 1, 3)
    g_issue(e + 3, 3)
    g_wait(e + 1, 1)
    s_issue(e + 1, 1)
    g_wait(e + 2, 2)
    s_issue(e + 2, 2)
    g_wait(e + 3, 3)
    s_issue(e + 3, 3)
    s_wait(e, 0)
    s_wait(e + 1, 1)
    s_wait(e + 2, 2)
    s_wait(e + 3, 3)
    plsc.subcore_barrier()

    _drain_band(s, c, acc_sh, out_hbm, D, D)

  return body


@functools.partial(
    pl.kernel,
    out_type=jax.ShapeDtypeStruct((N, 128), jnp.float32),
    mesh=_MESH,
    compiler_params=_SC_PARAMS,
    scratch_types=[
        pltpu.VMEM((NCHUNK, K), jnp.int32),
        pltpu.VMEM((K, 8), jnp.float32),
        pltpu.VMEM((ZR, 8), jnp.float32),
        pltpu.VMEM_SHARED((N, 8), jnp.float32),
        [pltpu.SemaphoreType.DMA] * 4,
    ],
)
def _sc_degree_body(dst_hbm, ones_hbm, ztile_hbm, out_hbm, dst_all, ones_v,
                    zbuf, acc_sh, ssems):
  c = lax.axis_index("c")
  s = lax.axis_index("s")
  wid = c * NS + s

  pltpu.sync_copy(dst_hbm.at[wid], dst_all)
  pltpu.sync_copy(ones_hbm, ones_v)
  pltpu.sync_copy(ztile_hbm, zbuf)
  _zero_band(s, zbuf, acc_sh, ssems[0])
  plsc.subcore_barrier()

  def s_issue(i, t):
    pltpu.async_copy(ones_v, acc_sh.at[dst_all.at[i]], ssems[t], add=True)

  def s_wait(i, t):
    pltpu.make_async_copy(ones_v, acc_sh.at[dst_all.at[i]], ssems[t]).wait()

  s_issue(0, 0)
  s_issue(1, 1)
  s_issue(2, 2)
  s_issue(3, 3)

  def quad(jj, carry):
    base = jj * 4
    for t in range(4):
      i = base + t
      s_wait(i - 4, t)
      s_issue(i, t)
    return carry

  lax.fori_loop(1, NCHUNK // 4, quad, 0)
  for t in range(4):
    s_wait(NCHUNK - 4 + t, t)
  plsc.subcore_barrier()

  _drain_band(s, c, acc_sh, out_hbm, 8, 64)


def _scale_body(p_ref, x_ref, w_ref, dinv_ref, y_ref):
  dinv = lax.rsqrt(1.0 + p_ref[:, 0:1] + p_ref[:, 64:65])
  dinv_ref[...] = dinv
  xw = jnp.dot(x_ref[...], w_ref[...], preferred_element_type=jnp.float32)
  y_ref[...] = dinv * xw


def _tc_scale_mm(degparts, x, w):
  return pl.pallas_call(
      _scale_body,
      out_shape=[
          jax.ShapeDtypeStruct((N, 1), jnp.float32),
          jax.ShapeDtypeStruct((N, w.shape[1]), jnp.float32),
      ],
  )(degparts, x, w)


def _layer_body(a_ref, y_ref, dinv_ref, b_ref, w_ref, o_ref):
  d = w_ref.shape[0]
  t = a_ref[:, 0:d] + a_ref[:, d:2 * d] + y_ref[...]
  h = jnp.maximum(dinv_ref[...] * t + b_ref[...], 0.0)
  o_ref[...] = dinv_ref[...] * jnp.dot(h, w_ref[...],
                                       preferred_element_type=jnp.float32)


def _tc_layer(accparts, y, dinv, b, w):
  return pl.pallas_call(
      _layer_body,
      out_shape=jax.ShapeDtypeStruct((N, w.shape[1]), jnp.float32),
  )(accparts, y, dinv, b, w)


def _combine_body(a_ref, y_ref, dinv_ref, b_ref, o_ref):
  d = b_ref.shape[1]
  t = a_ref[:, 0:d] + a_ref[:, d:2 * d] + y_ref[...]
  o_ref[...] = dinv_ref[...] * t + b_ref[...]


def _tc_combine(accparts, y, dinv, b):
  d = b.shape[1]
  return pl.pallas_call(
      _combine_body,
      out_shape=jax.ShapeDtypeStruct((N, d), jnp.float32),
  )(accparts, y, dinv, b)


def kernel(x, edge_index, W1, b1, W2, b2):
  srcw = edge_index[0].reshape(NW, NCHUNK, K)
  dstw = edge_index[1].reshape(NW, NCHUNK, K)

  degparts = _sc_degree_body(dstw, jnp.ones((K, 8), jnp.float32),
                             jnp.zeros((ZR, 8), jnp.float32))
  dinv, y1 = _tc_scale_mm(degparts, x, W1)

  acc1 = _make_sc_accumulate(64)(y1, srcw, dstw,
                                 jnp.zeros((ZR, 64), jnp.float32))
  y2 = _tc_layer(acc1, y1, dinv, b1.reshape(1, -1), W2)

  acc2 = _make_sc_accumulate(32)(y2, srcw, dstw,
                                 jnp.zeros((ZR, 32), jnp.float32))
  return _tc_combine(acc2, y2, dinv, b2.reshape(1, -1))


# speedup vs baseline: 55.6810x; 1.0285x over previous
import functools

import jax
import jax.numpy as jnp
from jax import lax
from jax.experimental import pallas as pl
from jax.experimental.pallas import tpu as pltpu
from jax.experimental.pallas import tpu_sc as plsc

N = 10000
E = 320000
NC = 2
NS = 16
NW = NC * NS
EPW = E // NW
K = 125
NCHUNK = EPW // K
ZR = 80
RB = 640
RB_LAST = N - 15 * RB

_MESH = plsc.VectorSubcoreMesh(core_axis_name="c", subcore_axis_name="s")
_SC_PARAMS = pltpu.CompilerParams(use_tc_tiling_on_sc=False)


def _zero_band(s, zbuf, acc_sh, zsem):
  r0 = s * RB

  @pl.when(s < NS - 1)
  def _():
    for t in range(RB // ZR):
      pltpu.async_copy(zbuf, acc_sh.at[pl.ds(r0 + t * ZR, ZR)], zsem)
    for t in range(RB // ZR):
      pltpu.make_async_copy(zbuf, acc_sh.at[pl.ds(r0 + t * ZR, ZR)],
                            zsem).wait()

  @pl.when(s == NS - 1)
  def _():
    for t in range(RB_LAST // ZR):
      pltpu.async_copy(zbuf, acc_sh.at[pl.ds(r0 + t * ZR, ZR)], zsem)
    for t in range(RB_LAST // ZR):
      pltpu.make_async_copy(zbuf, acc_sh.at[pl.ds(r0 + t * ZR, ZR)],
                            zsem).wait()


def _drain_band(s, c, acc_sh, out_hbm, D, stride):
  r0 = s * RB

  @pl.when(s < NS - 1)
  def _():
    pltpu.sync_copy(acc_sh.at[pl.ds(r0, RB)],
                    out_hbm.at[pl.ds(r0, RB), pl.ds(c * stride, D)])

  @pl.when(s == NS - 1)
  def _():
    pltpu.sync_copy(acc_sh.at[pl.ds(r0, RB_LAST)],
                    out_hbm.at[pl.ds(r0, RB_LAST), pl.ds(c * stride, D)])


def _make_sc_accumulate(D):
  @functools.partial(
      pl.kernel,
      out_type=jax.ShapeDtypeStruct((N, 128), jnp.float32),
      mesh=_MESH,
      compiler_params=_SC_PARAMS,
      scratch_types=[
          pltpu.VMEM((NCHUNK, K), jnp.int32),
          pltpu.VMEM((NCHUNK, K), jnp.int32),
          pltpu.VMEM((8, K, D), jnp.float32),
          pltpu.VMEM((ZR, D), jnp.float32),
          pltpu.VMEM_SHARED((N, D), jnp.float32),
          [pltpu.SemaphoreType.DMA] * 8,
          [pltpu.SemaphoreType.DMA] * 8,
      ],
  )
  def body(y_hbm, src_hbm, dst_hbm, ztile_hbm, out_hbm, src_all, dst_all,
           ring, zbuf, acc_sh, gsems, ssems):
    c = lax.axis_index("c")
    s = lax.axis_index("s")
    wid = c * NS + s

    pltpu.sync_copy(src_hbm.at[wid], src_all)
    pltpu.sync_copy(dst_hbm.at[wid], dst_all)
    pltpu.sync_copy(ztile_hbm, zbuf)
    _zero_band(s, zbuf, acc_sh, gsems[0])
    plsc.subcore_barrier()

    def g_issue(i, t):
      pltpu.async_copy(y_hbm.at[src_all.at[i]], ring.at[t], gsems[t])

    def g_wait(i, t):
      pltpu.make_async_copy(y_hbm.at[src_all.at[i]], ring.at[t],
                            gsems[t]).wait()

    def s_issue(i, t):
      pltpu.async_copy(ring.at[t], acc_sh.at[dst_all.at[i]], ssems[t],
                       add=True)

    def s_wait(i, t):
      pltpu.make_async_copy(ring.at[t], acc_sh.at[dst_all.at[i]],
                            ssems[t]).wait()

    for t in range(4):
      g_issue(t, t)
    for t in range(4):
      g_issue(t + 4, t + 4)
      g_wait(t, t)
      s_issue(t, t)
    for t in range(4):
      s_wait(t, t)
      g_issue(t + 8, t)
      g_wait(t + 4, t + 4)
      s_issue(t + 4, t + 4)

    def oct_step(jj, carry):
      base = jj * 8
      for t in range(8):
        i = base + t
        s_wait(i - 4, (t + 4) % 8)
        g_issue(i + 4, (t + 4) % 8)
        g_wait(i, t)
        s_issue(i, t)
      return carry

    lax.fori_loop(1, NCHUNK // 8 - 1, oct_step, 0)

    e = NCHUNK - 8
    for t in range(4):
      s_wait(e - 4 + t, (t + 4) % 8)
      g_issue(e + 4 + t, (t + 4) % 8)
      g_wait(e + t, t)
      s_issue(e + t, t)
    for t in range(4, 8):
      g_wait(e + t, t)
      s_issue(e + t, t)
    for t in range(8):
      s_wait(e + t, t)
    plsc.subcore_barrier()

    _drain_band(s, c, acc_sh, out_hbm, D, D)

  return body


@functools.partial(
    pl.kernel,
    out_type=jax.ShapeDtypeStruct((N, 128), jnp.float32),
    mesh=_MESH,
    compiler_params=_SC_PARAMS,
    scratch_types=[
        pltpu.VMEM((NCHUNK, K), jnp.int32),
        pltpu.VMEM((K, 8), jnp.float32),
        pltpu.VMEM((ZR, 8), jnp.float32),
        pltpu.VMEM_SHARED((N, 8), jnp.float32),
        [pltpu.SemaphoreType.DMA] * 4,
    ],
)
def _sc_degree_body(dst_hbm, ones_hbm, ztile_hbm, out_hbm, dst_all, ones_v,
                    zbuf, acc_sh, ssems):
  c = lax.axis_index("c")
  s = lax.axis_index("s")
  wid = c * NS + s

  pltpu.sync_copy(dst_hbm.at[wid], dst_all)
  pltpu.sync_copy(ones_hbm, ones_v)
  pltpu.sync_copy(ztile_hbm, zbuf)
  _zero_band(s, zbuf, acc_sh, ssems[0])
  plsc.subcore_barrier()

  def s_issue(i, t):
    pltpu.async_copy(ones_v, acc_sh.at[dst_all.at[i]], ssems[t], add=True)

  def s_wait(i, t):
    pltpu.make_async_copy(ones_v, acc_sh.at[dst_all.at[i]], ssems[t]).wait()

  s_issue(0, 0)
  s_issue(1, 1)
  s_issue(2, 2)
  s_issue(3, 3)

  def quad(jj, carry):
    base = jj * 4
    for t in range(4):
      i = base + t
      s_wait(i - 4, t)
      s_issue(i, t)
    return carry

  lax.fori_loop(1, NCHUNK // 4, quad, 0)
  for t in range(4):
    s_wait(NCHUNK - 4 + t, t)
  plsc.subcore_barrier()

  _drain_band(s, c, acc_sh, out_hbm, 8, 64)


def _scale_body(p_ref, x_ref, w_ref, dinv_ref, y_ref):
  dinv = lax.rsqrt(1.0 + p_ref[:, 0:1] + p_ref[:, 64:65])
  dinv_ref[...] = dinv
  xw = jnp.dot(x_ref[...], w_ref[...], preferred_element_type=jnp.float32)
  y_ref[...] = dinv * xw


def _tc_scale_mm(degparts, x, w):
  return pl.pallas_call(
      _scale_body,
      out_shape=[
          jax.ShapeDtypeStruct((N, 1), jnp.float32),
          jax.ShapeDtypeStruct((N, w.shape[1]), jnp.float32),
      ],
  )(degparts, x, w)


def _layer_body(a_ref, y_ref, dinv_ref, b_ref, w_ref, o_ref):
  d = w_ref.shape[0]
  t = a_ref[:, 0:d] + a_ref[:, d:2 * d] + y_ref[...]
  h = jnp.maximum(dinv_ref[...] * t + b_ref[...], 0.0)
  o_ref[...] = dinv_ref[...] * jnp.dot(h, w_ref[...],
                                       preferred_element_type=jnp.float32)


def _tc_layer(accparts, y, dinv, b, w):
  return pl.pallas_call(
      _layer_body,
      out_shape=jax.ShapeDtypeStruct((N, w.shape[1]), jnp.float32),
  )(accparts, y, dinv, b, w)


def _combine_body(a_ref, y_ref, dinv_ref, b_ref, o_ref):
  d = b_ref.shape[1]
  t = a_ref[:, 0:d] + a_ref[:, d:2 * d] + y_ref[...]
  o_ref[...] = dinv_ref[...] * t + b_ref[...]


def _tc_combine(accparts, y, dinv, b):
  d = b.shape[1]
  return pl.pallas_call(
      _combine_body,
      out_shape=jax.ShapeDtypeStruct((N, d), jnp.float32),
  )(accparts, y, dinv, b)


def kernel(x, edge_index, W1, b1, W2, b2):
  srcw = edge_index[0].reshape(NW, NCHUNK, K)
  dstw = edge_index[1].reshape(NW, NCHUNK, K)

  degparts = _sc_degree_body(dstw, jnp.ones((K, 8), jnp.float32),
                             jnp.zeros((ZR, 8), jnp.float32))
  dinv, y1 = _tc_scale_mm(degparts, x, W1)

  acc1 = _make_sc_accumulate(64)(y1, srcw, dstw,
                                 jnp.zeros((ZR, 64), jnp.float32))
  y2 = _tc_layer(acc1, y1, dinv, b1.reshape(1, -1), W2)

  acc2 = _make_sc_accumulate(32)(y2, srcw, dstw,
                                 jnp.zeros((ZR, 32), jnp.float32))
  return _tc_combine(acc2, y2, dinv, b2.reshape(1, -1))
